# baseline (speedup 1.0000x reference)
import functools

import jax
import jax.numpy as jnp
from jax import lax
from jax.experimental import pallas as pl
from jax.experimental.pallas import tpu as pltpu
from jax.experimental.pallas import tpu_sc as plsc

_V = 128
_D = 49152
_B = 1024
_NC = 2
_NS = 16
_DH = _D // _NC
_BPT = _B // _NS
_C = 4096
_NCHUNK = _DH // _C
_RPT = _V // _NS
_L = 16


def _gather_body(table_hbm, idx_hbm, out_hbm,
                 idx_v, sbuf0, sbuf1, lsem0, lsem1, gsem):
    cid = lax.axis_index("c")
    sid = lax.axis_index("s")
    base = sid * _BPT
    chalf = cid * _DH
    pltpu.sync_copy(idx_hbm.at[pl.ds(base, _BPT)], idx_v)

    scalars = []
    for v in range(_BPT // _L):
        vec = idx_v[pl.ds(v * _L, _L)]
        for j in range(_L):
            scalars.append(vec[j])

    sbufs = (sbuf0, sbuf1)
    lsems = (lsem0, lsem1)
    row0 = sid * _RPT

    def load_desc(c, slot):
        off = chalf + pl.multiple_of(c * _C, _C)
        return pltpu.make_async_copy(
            table_hbm.at[pl.ds(row0, _RPT), pl.ds(off, _C)],
            sbufs[slot].at[pl.ds(row0, _RPT)],
            lsems[slot])

    def do_chunk(c, slot):
        load_desc(c, slot).wait()
        plsc.subcore_barrier()

        @pl.when(c + 1 < _NCHUNK)
        def _():
            load_desc(c + 1, 1 - slot).start()

        off = chalf + pl.multiple_of(c * _C, _C)
        copies = []
        for j in range(_BPT):
            cp = pltpu.make_async_copy(
                sbufs[slot].at[pl.ds(scalars[j], 1)],
                out_hbm.at[pl.ds(base + j, 1), pl.ds(off, _C)],
                gsem)
            cp.start()
            copies.append(cp)
        for cp in copies:
            cp.wait()

    load_desc(0, 0).start()

    def pair_body(p, carry):
        c = p * 2
        do_chunk(c, 0)
        do_chunk(c + 1, 1)
        return carry

    lax.fori_loop(0, _NCHUNK // 2, pair_body, 0)


@jax.jit
def _gather(table, idx):
    mesh = plsc.VectorSubcoreMesh(core_axis_name="c", subcore_axis_name="s")
    f = pl.kernel(
        _gather_body,
        out_type=jax.ShapeDtypeStruct((_B, _D), jnp.float32),
        mesh=mesh,
        scratch_types=[
            pltpu.VMEM((_BPT,), jnp.int32),
            pltpu.VMEM_SHARED((_V, _C), jnp.float32),
            pltpu.VMEM_SHARED((_V, _C), jnp.float32),
            pltpu.SemaphoreType.DMA,
            pltpu.SemaphoreType.DMA,
            pltpu.SemaphoreType.DMA,
        ],
    )
    return f(table, idx)


def kernel(prefix, embedding):
    idx = prefix.reshape(-1).astype(jnp.int32)
    out = _gather(embedding, idx)
    return out.reshape(prefix.shape[0], prefix.shape[1], _D)
